# baseline (speedup 1.0000x reference)
import functools

import jax
import jax.numpy as jnp
from jax import lax
from jax.experimental import pallas as pl
from jax.experimental.pallas import tpu as pltpu
from jax.experimental.pallas import tpu_sc as plsc

_F = 384
_C = 384
_K_SIZE = _F * _C
_P = 50176

_PB = 3584
_S = _P // _PB
_Q = 4
_RQ = _C // _Q
_D = 3
_O = 2

_NW = 32
_CHUNK_ROWS = 4
_LANES = 128
_NNZ_PAD = _NW * _CHUNK_ROWS * _LANES
_SLICE = _K_SIZE // 16


def _scatter_body(idx_hbm, val_hbm, zero_hbm, out_hbm, idx_v, val_v, kacc):
    cid = lax.axis_index("c")
    sid = lax.axis_index("s")
    g = sid * 2 + cid
    pltpu.sync_copy(zero_hbm.at[pl.ds(sid * _SLICE, _SLICE)],
                    kacc.at[pl.ds(sid * _SLICE, _SLICE)])
    pltpu.sync_copy(idx_hbm.at[g], idx_v)
    pltpu.sync_copy(val_hbm.at[g], val_v)
    plsc.subcore_barrier()
    for j in range(_CHUNK_ROWS):
        pltpu.sync_copy(val_v.at[j], kacc.at[idx_v.at[j]], add=True)
    plsc.subcore_barrier()
    pltpu.sync_copy(kacc.at[pl.ds(sid * _SLICE, _SLICE)],
                    out_hbm.at[cid, pl.ds(sid * _SLICE, _SLICE)])


@jax.jit
def _build_kernel_coo(values, row_ids, col_ids):
    flat_idx = row_ids * _C + col_ids
    pad = _NNZ_PAD - values.shape[0]
    idx = jnp.concatenate([flat_idx, jnp.zeros((pad,), jnp.int32)])
    val = jnp.concatenate([values, jnp.zeros((pad,), jnp.float32)])
    idx = idx.reshape(_NW, _CHUNK_ROWS, _LANES)
    val = val.reshape(_NW, _CHUNK_ROWS, _LANES)
    zero = jnp.zeros((_K_SIZE,), jnp.float32)
    mesh = plsc.VectorSubcoreMesh(core_axis_name="c", subcore_axis_name="s")
    fn = functools.partial(
        pl.kernel,
        mesh=mesh,
        out_type=jax.ShapeDtypeStruct((2, _K_SIZE), jnp.float32),
        scratch_types=[
            pltpu.VMEM((_CHUNK_ROWS, _LANES), jnp.int32),
            pltpu.VMEM((_CHUNK_ROWS, _LANES), jnp.float32),
            pltpu.VMEM_SHARED((_K_SIZE,), jnp.float32),
        ],
    )(_scatter_body)
    return fn(idx, val, zero)


def _in_copy(x_ref, xbuf, step, slot, q, sem):
    return pltpu.make_async_copy(
        x_ref.at[pl.ds(q * _RQ, _RQ), pl.ds(step * _PB, _PB)],
        xbuf.at[slot, pl.ds(q * _RQ, _RQ)],
        sem,
    )


def _out_copy(o_ref, obuf, step, slot, q, sem):
    return pltpu.make_async_copy(
        obuf.at[slot, pl.ds(q * _RQ, _RQ)],
        o_ref.at[pl.ds(q * _RQ, _RQ), pl.ds(step * _PB, _PB)],
        sem,
    )


def _mm_body(k_ref, x_ref, o_ref, xbuf, obuf, *sems):
    in_sems = sems[:_D]
    out_sems = sems[_D:]
    i = pl.program_id(0)

    @pl.when(i == 0)
    def _prologue():
        for d in range(_D - 1):
            for q in range(_Q):
                _in_copy(x_ref, xbuf, d, d, q, in_sems[d]).start()

    @pl.when(i + _D - 1 < _S)
    def _issue_ahead():
        step = i + _D - 1
        for d in range(_D):
            @pl.when(step % _D == d)
            def _issue(d=d, step=step):
                for q in range(_Q):
                    _in_copy(x_ref, xbuf, step, d, q, in_sems[d]).start()

    for d in range(_D):
        @pl.when(i % _D == d)
        def _wait_in(d=d):
            for q in range(_Q):
                _in_copy(x_ref, xbuf, i, d, q, in_sems[d]).wait()

    @pl.when(i >= _O)
    def _wait_out_slot():
        for o in range(_O):
            @pl.when(i % _O == o)
            def _wait(o=o):
                for q in range(_Q):
                    _out_copy(o_ref, obuf, i - _O, o, q, out_sems[o]).wait()

    kb = (k_ref[0] + k_ref[1]).astype(jnp.bfloat16)
    for d in range(_D):
        @pl.when(i % _D == d)
        def _compute(d=d):
            res = jax.lax.dot_general(
                kb, xbuf[d].astype(jnp.bfloat16),
                dimension_numbers=(((1,), (0,)), ((), ())),
                preferred_element_type=jnp.float32,
            )
            for o in range(_O):
                @pl.when(i % _O == o)
                def _store(o=o, res=res):
                    obuf[o] = res

    for o in range(_O):
        @pl.when(i % _O == o)
        def _issue_out(o=o):
            for q in range(_Q):
                _out_copy(o_ref, obuf, i, o, q, out_sems[o]).start()

    @pl.when(i == _S - 1)
    def _epilogue():
        for back in range(_O):
            step = _S - _O + back
            for o in range(_O):
                @pl.when(step % _O == o)
                def _drain(o=o, step=step):
                    for q in range(_Q):
                        _out_copy(o_ref, obuf, step, o, q, out_sems[o]).wait()


@jax.jit
def _matmul(kparts, x):
    return pl.pallas_call(
        _mm_body,
        grid=(_S,),
        in_specs=[
            pl.BlockSpec((2, _F, _C), lambda i: (0, 0, 0)),
            pl.BlockSpec(memory_space=pltpu.MemorySpace.HBM),
        ],
        out_specs=pl.BlockSpec(memory_space=pltpu.MemorySpace.HBM),
        out_shape=jax.ShapeDtypeStruct((_F, _P), jnp.float32),
        scratch_shapes=[
            pltpu.VMEM((_D, _C, _PB), jnp.float32),
            pltpu.VMEM((_O, _F, _PB), jnp.float32),
        ] + [pltpu.SemaphoreType.DMA] * (_D + _O),
        compiler_params=pltpu.CompilerParams(
            dimension_semantics=("arbitrary",),
        ),
    )(kparts, x)


def kernel(inputs, values, row_ids, col_ids):
    b, c, h, w = inputs.shape
    kparts = _build_kernel_coo(values, row_ids, col_ids).reshape(2, _F, _C)
    flat = inputs.reshape(c, h * w)
    out = _matmul(kparts, flat)
    return out.reshape(b, _F, h, w)
